# Initial kernel scaffold; baseline (speedup 1.0000x reference)
import jax
import jax.numpy as jnp
from jax.experimental import pallas as pl


def kernel(x_article, x_source, edge_index_pb, edge_index_sd, W1l, b1l, W1r, Wsl, bsl, Wsr, W2l, b2l, W2r):
    raise NotImplementedError("write your pallas kernel here")


# speedup vs baseline: 3.1752x; 3.1752x over previous
import jax
import jax.numpy as jnp
from jax import lax
from jax.experimental import pallas as pl
from jax.experimental.pallas import tpu as pltpu
from jax.experimental.pallas import tpu_sc as plsc

N_ART = 10000
D = 128
NC, NS = 2, 16
NW = NC * NS
N_PAD = 10240
RPS = N_PAD // NS
CH = 128
SD_CHUNKS = 79
PB_CHUNKS = 3
E_SD_PAD = NW * SD_CHUNKS * CH
E_PB_PAD = NW * PB_CHUNKS * CH


def _make_sum_call(n_chunks):
    def body(table, src_i, dst_i, o_acc, acc, idx_s, idx_d, rows, zrow, sem):
        c = lax.axis_index("c")
        s = lax.axis_index("s")
        wid = s * NC + c
        rbase = s * RPS
        zv = jnp.zeros((16,), jnp.float32)

        def _fz(i, carry):
            for j in range(D // 16):
                zrow[i, pl.ds(j * 16, 16)] = zv
            return carry
        lax.fori_loop(0, zrow.shape[0], _fz, 0)

        def zb(j, carry):
            pltpu.sync_copy(zrow, acc.at[pl.ds(rbase + j * 16, 16)])
            return carry
        lax.fori_loop(0, RPS // 16, zb, 0)

        plsc.subcore_barrier()
        ebase = wid * (n_chunks * CH)

        def eb(i, carry):
            off = pl.multiple_of(ebase + i * CH, 8)
            pltpu.sync_copy(src_i.at[pl.ds(off, CH)], idx_s)
            pltpu.sync_copy(dst_i.at[pl.ds(off, CH)], idx_d)
            pltpu.async_copy(table.at[idx_s], rows, sem).wait()
            pltpu.sync_copy(rows, acc.at[idx_d], add=True)
            return carry
        lax.fori_loop(0, n_chunks, eb, 0)

        plsc.subcore_barrier()
        pltpu.sync_copy(acc.at[pl.ds(rbase, RPS)],
                        o_acc.at[c, pl.ds(rbase, RPS)])

    f32 = jnp.float32
    mesh = plsc.VectorSubcoreMesh(core_axis_name="c", subcore_axis_name="s")
    return pl.kernel(
        body,
        out_type=[jax.ShapeDtypeStruct((NC, N_PAD, D), f32)],
        scratch_types=[
            pltpu.VMEM_SHARED((N_PAD, D), f32),
            pltpu.VMEM((CH,), jnp.int32),
            pltpu.VMEM((CH,), jnp.int32),
            pltpu.VMEM((CH, D), f32),
            pltpu.VMEM((16, D), f32),
            pltpu.SemaphoreType.DMA,
        ],
        mesh=mesh,
    )


def _make_cnt_call(n_chunks):
    def body(dst_i, o_acc, acc, idx_d, ones_rows, zrow, sem):
        c = lax.axis_index("c")
        s = lax.axis_index("s")
        wid = s * NC + c
        rbase = s * RPS
        zv = jnp.zeros((16,), jnp.float32)
        ov = jnp.ones((16,), jnp.float32)

        def _fz(i, carry):
            for j in range(D // 16):
                zrow[i, pl.ds(j * 16, 16)] = zv
            return carry
        lax.fori_loop(0, zrow.shape[0], _fz, 0)

        def _fo(i, carry):
            for j in range(D // 16):
                ones_rows[i, pl.ds(j * 16, 16)] = ov
            return carry
        lax.fori_loop(0, ones_rows.shape[0], _fo, 0)

        def zb(j, carry):
            pltpu.sync_copy(zrow, acc.at[pl.ds(rbase + j * 16, 16)])
            return carry
        lax.fori_loop(0, RPS // 16, zb, 0)

        plsc.subcore_barrier()
        ebase = wid * (n_chunks * CH)

        def eb(i, carry):
            off = pl.multiple_of(ebase + i * CH, 8)
            pltpu.sync_copy(dst_i.at[pl.ds(off, CH)], idx_d)
            pltpu.sync_copy(ones_rows, acc.at[idx_d], add=True)
            return carry
        lax.fori_loop(0, n_chunks, eb, 0)

        plsc.subcore_barrier()
        pltpu.sync_copy(acc.at[pl.ds(rbase, RPS)],
                        o_acc.at[c, pl.ds(rbase, RPS)])

    f32 = jnp.float32
    mesh = plsc.VectorSubcoreMesh(core_axis_name="c", subcore_axis_name="s")
    return pl.kernel(
        body,
        out_type=[jax.ShapeDtypeStruct((NC, N_PAD, D), f32)],
        scratch_types=[
            pltpu.VMEM_SHARED((N_PAD, D), f32),
            pltpu.VMEM((CH,), jnp.int32),
            pltpu.VMEM((CH, D), f32),
            pltpu.VMEM((16, D), f32),
            pltpu.SemaphoreType.DMA,
        ],
        mesh=mesh,
    )


BLK = 1024


def _tc_body(ssd, csd, spb, cpb, x,
             w1l, w1r, wsl, wsr, w2l, w2r, b1, bs, b2, o):
    def dot(a, b):
        return lax.dot_general(a, b[...], (((1,), (0,)), ((), ())),
                               preferred_element_type=jnp.float32)

    agg_sd = (ssd[0] + ssd[1]) / jnp.maximum(csd[0][:, 0:1] + csd[1][:, 0:1], 1.0)
    agg_pb = (spb[0] + spb[1]) / jnp.maximum(cpb[0][:, 0:1] + cpb[1][:, 0:1], 1.0)
    xb = x[...]
    h1 = jnp.maximum(dot(agg_pb, w1l) + b1[...] + dot(xb, w1r), 0.0)
    h2 = jnp.maximum(dot(agg_sd, wsl) + bs[...] + dot(xb, wsr), 0.0)
    o[...] = dot(agg_pb, w2l) + b2[...] + dot(h1 + h2, w2r)


def _tc_combine(ssd, csd, spb, cpb, x_art_p,
                w1lT, w1rT, wslT, wsrT, w2lT, w2rT, b1, bs, b2):
    f32 = jnp.float32
    sum_spec = pl.BlockSpec((NC, BLK, D), lambda i: (0, i, 0))
    w_spec = pl.BlockSpec((D, D), lambda i: (0, 0))
    b_spec = pl.BlockSpec((1, D), lambda i: (0, 0))
    return pl.pallas_call(
        _tc_body,
        grid=(N_PAD // BLK,),
        in_specs=[sum_spec, sum_spec, sum_spec, sum_spec,
                  pl.BlockSpec((BLK, D), lambda i: (i, 0)),
                  w_spec, w_spec, w_spec, w_spec, w_spec, w_spec,
                  b_spec, b_spec, b_spec],
        out_specs=pl.BlockSpec((BLK, D), lambda i: (i, 0)),
        out_shape=jax.ShapeDtypeStruct((N_PAD, D), f32),
    )(ssd, csd, spb, cpb, x_art_p,
      w1lT, w1rT, wslT, wsrT, w2lT, w2rT, b1, bs, b2)


def kernel(x_article, x_source, edge_index_pb, edge_index_sd,
           W1l, b1l, W1r, Wsl, bsl, Wsr, W2l, b2l, W2r):
    i32 = jnp.int32
    x_art_p = jnp.pad(x_article, ((0, N_PAD - N_ART), (0, 0)))
    src_sd = edge_index_sd[0].astype(i32)
    dst_sd = edge_index_sd[1].astype(i32)
    src_pb = edge_index_pb[1].astype(i32)
    dst_pb = edge_index_pb[0].astype(i32)

    pad_sd = E_SD_PAD - src_sd.shape[0]
    src_sd = jnp.pad(src_sd, (0, pad_sd))
    dst_sd = jnp.pad(dst_sd, (0, pad_sd), constant_values=N_PAD - 1)
    pad_pb = E_PB_PAD - src_pb.shape[0]
    src_pb = jnp.pad(src_pb, (0, pad_pb))
    dst_pb = jnp.pad(dst_pb, (0, pad_pb), constant_values=N_PAD - 1)

    (ssd,) = _make_sum_call(SD_CHUNKS)(x_art_p, src_sd, dst_sd)
    (spb,) = _make_sum_call(PB_CHUNKS)(x_source, src_pb, dst_pb)
    (csd,) = _make_cnt_call(SD_CHUNKS)(dst_sd)
    (cpb,) = _make_cnt_call(PB_CHUNKS)(dst_pb)

    out = _tc_combine(ssd, csd, spb, cpb, x_art_p,
                      W1l.T, W1r.T, Wsl.T, Wsr.T, W2l.T, W2r.T,
                      b1l.reshape(1, -1), bsl.reshape(1, -1),
                      b2l.reshape(1, -1))
    return out[:N_ART]
